# baseline (speedup 1.0000x reference)
import jax
import jax.numpy as jnp
from jax.experimental import pallas as pl
from jax.experimental.pallas import tpu as pltpu

_M = 2048


def _tile_body(ids_ref, x_ref, w_ref, out_ref):
    x = x_ref[...].astype(jnp.bfloat16)
    ids = ids_ref[0, 0, :]
    n_sub = 8
    in_f = x.shape[1]
    ids16 = ids.astype(jnp.int16)
    x2 = jnp.concatenate([x, x], axis=1)
    idsb2 = jnp.broadcast_to(ids16[:, None], (x.shape[0], 2 * in_f))
    lane_sub = (jax.lax.broadcasted_iota(jnp.int16, (1, 2 * in_f), 1)
                >= jnp.int16(in_f)).astype(jnp.int16)
    zero2 = jnp.zeros_like(x2)
    blocks = [jnp.where(idsb2 == lane_sub + jnp.int16(2 * k), x2, zero2)
              for k in range(n_sub // 2)]
    oh = (ids16[:, None] == jax.lax.broadcasted_iota(jnp.int16, (1, n_sub), 1)
          ).astype(jnp.bfloat16)
    xp = jnp.concatenate(blocks + [oh], axis=1)
    out_ref[...] = jnp.dot(xp, w_ref[...], preferred_element_type=jnp.float32)


def kernel(feat, subdet_id, proj_w, proj_b, type_table):
    n, in_f = feat.shape
    n_sub, embed = type_table.shape
    ids3 = subdet_id.reshape(n // _M, 1, _M)
    w2 = proj_w.reshape(n_sub * in_f, embed)
    tb = proj_b + type_table
    w3 = jnp.concatenate([w2, tb], axis=0).astype(jnp.bfloat16)
    return pl.pallas_call(
        _tile_body,
        grid=(n // _M,),
        in_specs=[
            pl.BlockSpec((1, 1, _M), lambda i: (i, 0, 0)),
            pl.BlockSpec((_M, in_f), lambda i: (i, 0)),
            pl.BlockSpec((n_sub * in_f + n_sub, embed), lambda i: (0, 0)),
        ],
        out_specs=pl.BlockSpec((_M, embed), lambda i: (i, 0)),
        out_shape=jax.ShapeDtypeStruct((n, embed), jnp.float32),
        compiler_params=pltpu.CompilerParams(
            dimension_semantics=("parallel",)),
    )(ids3, feat, w3)


# speedup vs baseline: 1.1718x; 1.1718x over previous
import jax
import jax.numpy as jnp
from jax.experimental import pallas as pl
from jax.experimental.pallas import tpu as pltpu

_M = 8192


def _tile_body(ids_ref, x_ref, w_ref, out_ref):
    x = x_ref[...].astype(jnp.bfloat16)
    ids = ids_ref[0, 0, :]
    n_sub = 8
    in_f = x.shape[1]
    ids16 = ids.astype(jnp.int16)
    x2 = jnp.concatenate([x, x], axis=1)
    idsb2 = jnp.broadcast_to(ids16[:, None], (x.shape[0], 2 * in_f))
    lane_sub = (jax.lax.broadcasted_iota(jnp.int16, (1, 2 * in_f), 1)
                >= jnp.int16(in_f)).astype(jnp.int16)
    zero2 = jnp.zeros_like(x2)
    blocks = [jnp.where(idsb2 == lane_sub + jnp.int16(2 * k), x2, zero2)
              for k in range(n_sub // 2)]
    oh = (ids16[:, None] == jax.lax.broadcasted_iota(jnp.int16, (1, n_sub), 1)
          ).astype(jnp.bfloat16)
    xp = jnp.concatenate(blocks + [oh], axis=1)
    out_ref[...] = jnp.dot(xp, w_ref[...], preferred_element_type=jnp.float32)


def kernel(feat, subdet_id, proj_w, proj_b, type_table):
    n, in_f = feat.shape
    n_sub, embed = type_table.shape
    ids3 = subdet_id.reshape(n // _M, 1, _M)
    w2 = proj_w.reshape(n_sub * in_f, embed)
    tb = proj_b + type_table
    w3 = jnp.concatenate([w2, tb], axis=0).astype(jnp.bfloat16)
    return pl.pallas_call(
        _tile_body,
        grid=(n // _M,),
        in_specs=[
            pl.BlockSpec((1, 1, _M), lambda i: (i, 0, 0)),
            pl.BlockSpec((_M, in_f), lambda i: (i, 0)),
            pl.BlockSpec((n_sub * in_f + n_sub, embed), lambda i: (0, 0)),
        ],
        out_specs=pl.BlockSpec((_M, embed), lambda i: (i, 0)),
        out_shape=jax.ShapeDtypeStruct((n, embed), jnp.float32),
        compiler_params=pltpu.CompilerParams(
            dimension_semantics=("parallel",), vmem_limit_bytes=100663296),
    )(ids3, feat, w3)
